# baseline (speedup 1.0000x reference)
import functools
import jax
import jax.numpy as jnp
from jax import lax
from jax.experimental import pallas as pl
from jax.experimental.pallas import tpu as pltpu
from jax.experimental.pallas import tpu_sc as plsc

_K = 8
_L = 16


def _logits_body(x_ref, w_ref, out_ref):
    logits = jnp.dot(x_ref[...], w_ref[...], preferred_element_type=jnp.float32)
    out_ref[...] = logits.T


def _fused_body(x_ref, w_ref, wprev_ref, iprev_ref, out_w_ref, out_i_ref):
    del wprev_ref, iprev_ref
    logits = jnp.dot(x_ref[...], w_ref[...], preferred_element_type=jnp.float32)
    bt, E = logits.shape
    lt = logits.T
    m = jnp.max(lt, axis=0, keepdims=True)
    e = jnp.exp(lt - m)
    z = jnp.sum(e, axis=0, keepdims=True)
    p = e / z

    iota_e = lax.broadcasted_iota(jnp.int32, (E, bt), 0)
    vals = []
    idxs = []
    cur = p
    for _ in range(_K):
        mk = jnp.max(cur, axis=0, keepdims=True)
        hit = cur == mk
        ik = jnp.min(jnp.where(hit, iota_e, E), axis=0, keepdims=True)
        vals.append(mk)
        idxs.append(ik)
        cur = jnp.where(iota_e == ik, -jnp.inf, cur)

    v = jnp.concatenate(vals, axis=0)
    i = jnp.concatenate(idxs, axis=0)
    e2 = jnp.exp(v - v[:1])
    w = e2 / jnp.sum(e2, axis=0, keepdims=True)
    out_w_ref[...] = w
    out_i_ref[...] = i


def _make_sc_router(ntok_total, E, BT):
    info = plsc.get_sparse_core_info()
    NW = info.num_cores * info.num_subcores
    ntok = ntok_total // NW
    ngrp = ntok // _L
    mesh = plsc.VectorSubcoreMesh(core_axis_name="c", subcore_axis_name="s")

    @functools.partial(
        pl.kernel,
        mesh=mesh,
        out_type=[
            jax.ShapeDtypeStruct((_K, BT), jnp.float32),
            jax.ShapeDtypeStruct((_K, BT), jnp.int32),
        ],
        scratch_types=[
            pltpu.VMEM((E, ntok), jnp.float32),
            pltpu.VMEM((E * _L,), jnp.float32),
            pltpu.VMEM((_K, ntok), jnp.float32),
            pltpu.VMEM((_K, ntok), jnp.int32),
        ],
        compiler_params=pltpu.CompilerParams(needs_layout_passes=False),
    )
    def sc_router(lt_hbm, out_w_hbm, out_i_hbm, lbuf, pbuf, wbuf, ibuf):
        wid = lax.axis_index("s") * info.num_cores + lax.axis_index("c")
        base = wid * ntok
        pltpu.sync_copy(lt_hbm.at[:, pl.ds(base, ntok)], lbuf)
        lane = lax.iota(jnp.int32, _L)

        ngroups8 = E // 8
        cvec = [jnp.int32(j * _L) + lane for j in range(8)]

        def group_body(g, carry):
            gbase = g * _L
            l = [lbuf[e, pl.ds(gbase, _L)] for e in range(E)]
            m = l[0]
            for e in range(1, E):
                m = jnp.maximum(m, l[e])
            ev = [jnp.exp(l[e] - m) for e in range(E)]
            z = ev[0]
            for e in range(1, E):
                z = z + ev[e]
            r = 1.0 / z
            p = [ev[e] * r for e in range(E)]
            for e in range(E):
                pbuf[pl.ds(e * _L, _L)] = p[e]

            gmax = []
            for gg in range(ngroups8):
                mg = p[gg * 8]
                for j in range(1, 8):
                    mg = jnp.maximum(mg, p[gg * 8 + j])
                gmax.append(mg)

            vk = []
            ik = []
            for k in range(_K):
                gv = list(gmax)
                gi = [jnp.full((_L,), gg, jnp.int32) for gg in range(ngroups8)]
                n = ngroups8
                while n > 1:
                    nv, ni = [], []
                    for j in range(0, n, 2):
                        ge = gv[j] >= gv[j + 1]
                        nv.append(jnp.where(ge, gv[j], gv[j + 1]))
                        ni.append(jnp.where(ge, gi[j], gi[j + 1]))
                    gv, gi, n = nv, ni, n // 2
                gwin = gi[0]
                base_flat = gwin * (8 * _L)
                mem = [
                    plsc.load_gather(pbuf, [base_flat + cvec[j]])
                    for j in range(8)
                ]
                mv = list(mem)
                mi = [jnp.full((_L,), j, jnp.int32) for j in range(8)]
                n = 8
                while n > 1:
                    nv, ni = [], []
                    for j in range(0, n, 2):
                        ge = mv[j] >= mv[j + 1]
                        nv.append(jnp.where(ge, mv[j], mv[j + 1]))
                        ni.append(jnp.where(ge, mi[j], mi[j + 1]))
                    mv, mi, n = nv, ni, n // 2
                jwin = mi[0]
                vk.append(mv[0])
                ik.append(gwin * 8 + jwin)
                plsc.store_scatter(
                    pbuf,
                    [base_flat + jwin * _L + lane],
                    jnp.full((_L,), -1.0, jnp.float32),
                )
                newm = jnp.where(jwin == 0, -1.0, mem[0])
                for j in range(1, 8):
                    newm = jnp.maximum(
                        newm, jnp.where(jwin == j, -1.0, mem[j])
                    )
                gmax = [
                    jnp.where(gwin == gg, newm, gmax[gg])
                    for gg in range(ngroups8)
                ]

            e2 = [jnp.exp(vk[k] - vk[0]) for k in range(_K)]
            s2 = e2[0]
            for k in range(1, _K):
                s2 = s2 + e2[k]
            r2 = 1.0 / s2
            for k in range(_K):
                wbuf[k, pl.ds(gbase, _L)] = e2[k] * r2
                ibuf[k, pl.ds(gbase, _L)] = ik[k]
            return carry

        lax.fori_loop(0, ngrp, group_body, 0)
        pltpu.sync_copy(wbuf, out_w_hbm.at[:, pl.ds(base, ntok)])
        pltpu.sync_copy(ibuf, out_i_hbm.at[:, pl.ds(base, ntok)])

    return sc_router


def kernel(x, kernel_DE):
    B, T, D = x.shape
    E = kernel_DE.shape[1]
    BT = B * T
    bt = 2048
    chunk = 12288
    nchunk = 1
    tail = BT - nchunk * chunk
    x2 = x.reshape(BT, D)
    sc_router = _make_sc_router(chunk, E, BT)

    lt = pl.pallas_call(
        _logits_body,
        grid=(chunk // bt,),
        in_specs=[
            pl.BlockSpec((bt, D), lambda i: (i, 0)),
            pl.BlockSpec((D, E), lambda i: (0, 0)),
        ],
        out_specs=pl.BlockSpec((E, bt), lambda i: (0, i)),
        out_shape=jax.ShapeDtypeStruct((E, chunk), jnp.float32),
    )(x2, kernel_DE)
    w_sc, i_sc = sc_router(lt)

    tb = chunk // bt
    w_t, i_t = pl.pallas_call(
        _fused_body,
        grid=(tail // bt,),
        in_specs=[
            pl.BlockSpec((bt, D), lambda i: (tb + i, 0)),
            pl.BlockSpec((D, E), lambda i: (0, 0)),
            pl.BlockSpec((_K, bt), lambda i: (0, tb + i)),
            pl.BlockSpec((_K, bt), lambda i: (0, tb + i)),
        ],
        out_specs=[
            pl.BlockSpec((_K, bt), lambda i: (0, tb + i)),
            pl.BlockSpec((_K, bt), lambda i: (0, tb + i)),
        ],
        out_shape=[
            jax.ShapeDtypeStruct((_K, BT), jnp.float32),
            jax.ShapeDtypeStruct((_K, BT), jnp.int32),
        ],
        input_output_aliases={2: 0, 3: 1},
    )(x2, kernel_DE, w_sc, i_sc)

    return w_t.T.reshape(B, T, _K), i_t.T.reshape(B, T, _K)


# speedup vs baseline: 1.1694x; 1.1694x over previous
import functools
import jax
import jax.numpy as jnp
from jax import lax
from jax.experimental import pallas as pl
from jax.experimental.pallas import tpu as pltpu
from jax.experimental.pallas import tpu_sc as plsc

_K = 8
_L = 16


def _logits_body(x_ref, w_ref, out_ref):
    logits = jnp.dot(x_ref[...], w_ref[...], preferred_element_type=jnp.float32)
    out_ref[...] = logits.T


def _fused_body(x_ref, w_ref, out_w_ref, out_i_ref):
    logits = jnp.dot(x_ref[...], w_ref[...], preferred_element_type=jnp.float32)
    bt, E = logits.shape
    lt = logits.T
    m = jnp.max(lt, axis=0, keepdims=True)
    e = jnp.exp(lt - m)
    z = jnp.sum(e, axis=0, keepdims=True)
    p = e / z

    iota_e = lax.broadcasted_iota(jnp.int32, (E, bt), 0)
    vals = []
    idxs = []
    cur = p
    for _ in range(_K):
        mk = jnp.max(cur, axis=0, keepdims=True)
        hit = cur == mk
        ik = jnp.min(jnp.where(hit, iota_e, E), axis=0, keepdims=True)
        vals.append(mk)
        idxs.append(ik)
        cur = jnp.where(iota_e == ik, -jnp.inf, cur)

    v = jnp.concatenate(vals, axis=0)
    i = jnp.concatenate(idxs, axis=0)
    e2 = jnp.exp(v - v[:1])
    w = e2 / jnp.sum(e2, axis=0, keepdims=True)
    out_w_ref[...] = w
    out_i_ref[...] = i


def _make_sc_router(ntok_total, E, BT):
    info = plsc.get_sparse_core_info()
    NW = info.num_cores * info.num_subcores
    ntok = ntok_total // NW
    ngrp = ntok // _L
    mesh = plsc.VectorSubcoreMesh(core_axis_name="c", subcore_axis_name="s")

    @functools.partial(
        pl.kernel,
        mesh=mesh,
        out_type=[
            jax.ShapeDtypeStruct((_K, ntok_total), jnp.float32),
            jax.ShapeDtypeStruct((_K, ntok_total), jnp.int32),
        ],
        scratch_types=[
            pltpu.VMEM((E, ntok), jnp.float32),
            pltpu.VMEM((E * _L,), jnp.float32),
            pltpu.VMEM((_K, ntok), jnp.float32),
            pltpu.VMEM((_K, ntok), jnp.int32),
        ],
        compiler_params=pltpu.CompilerParams(needs_layout_passes=False),
    )
    def sc_router(lt_hbm, out_w_hbm, out_i_hbm, lbuf, pbuf, wbuf, ibuf):
        wid = lax.axis_index("s") * info.num_cores + lax.axis_index("c")
        base = wid * ntok
        pltpu.sync_copy(lt_hbm.at[:, pl.ds(base, ntok)], lbuf)
        lane = lax.iota(jnp.int32, _L)

        ngroups8 = E // 8
        cvec = [jnp.int32(j * _L) + lane for j in range(8)]

        def group_body(g, carry):
            gbase = g * _L
            l = [lbuf[e, pl.ds(gbase, _L)] for e in range(E)]
            m = l[0]
            for e in range(1, E):
                m = jnp.maximum(m, l[e])
            ev = [jnp.exp(l[e] - m) for e in range(E)]
            z = ev[0]
            for e in range(1, E):
                z = z + ev[e]
            r = 1.0 / z
            p = [ev[e] * r for e in range(E)]
            for e in range(E):
                pbuf[pl.ds(e * _L, _L)] = p[e]

            gmax = []
            for gg in range(ngroups8):
                mg = p[gg * 8]
                for j in range(1, 8):
                    mg = jnp.maximum(mg, p[gg * 8 + j])
                gmax.append(mg)

            vk = []
            ik = []
            for k in range(_K):
                gv = list(gmax)
                gi = [jnp.full((_L,), gg, jnp.int32) for gg in range(ngroups8)]
                n = ngroups8
                while n > 1:
                    nv, ni = [], []
                    for j in range(0, n, 2):
                        ge = gv[j] >= gv[j + 1]
                        nv.append(jnp.where(ge, gv[j], gv[j + 1]))
                        ni.append(jnp.where(ge, gi[j], gi[j + 1]))
                    gv, gi, n = nv, ni, n // 2
                gwin = gi[0]
                base_flat = gwin * (8 * _L)
                mem = [
                    plsc.load_gather(pbuf, [base_flat + cvec[j]])
                    for j in range(8)
                ]
                mv = list(mem)
                mi = [jnp.full((_L,), j, jnp.int32) for j in range(8)]
                n = 8
                while n > 1:
                    nv, ni = [], []
                    for j in range(0, n, 2):
                        ge = mv[j] >= mv[j + 1]
                        nv.append(jnp.where(ge, mv[j], mv[j + 1]))
                        ni.append(jnp.where(ge, mi[j], mi[j + 1]))
                    mv, mi, n = nv, ni, n // 2
                jwin = mi[0]
                vk.append(mv[0])
                ik.append(gwin * 8 + jwin)
                plsc.store_scatter(
                    pbuf,
                    [base_flat + jwin * _L + lane],
                    jnp.full((_L,), -1.0, jnp.float32),
                )
                newm = jnp.where(jwin == 0, -1.0, mem[0])
                for j in range(1, 8):
                    newm = jnp.maximum(
                        newm, jnp.where(jwin == j, -1.0, mem[j])
                    )
                gmax = [
                    jnp.where(gwin == gg, newm, gmax[gg])
                    for gg in range(ngroups8)
                ]

            e2 = [jnp.exp(vk[k] - vk[0]) for k in range(_K)]
            s2 = e2[0]
            for k in range(1, _K):
                s2 = s2 + e2[k]
            r2 = 1.0 / s2
            for k in range(_K):
                wbuf[k, pl.ds(gbase, _L)] = e2[k] * r2
                ibuf[k, pl.ds(gbase, _L)] = ik[k]
            return carry

        lax.fori_loop(0, ngrp, group_body, 0)
        pltpu.sync_copy(wbuf, out_w_hbm.at[:, pl.ds(base, ntok)])
        pltpu.sync_copy(ibuf, out_i_hbm.at[:, pl.ds(base, ntok)])

    return sc_router


def kernel(x, kernel_DE):
    B, T, D = x.shape
    E = kernel_DE.shape[1]
    BT = B * T
    bt = 2048
    chunk = 12288
    nchunk = 1
    tail = BT - nchunk * chunk
    x2 = x.reshape(BT, D)
    sc_router = _make_sc_router(chunk, E, BT)

    lt = pl.pallas_call(
        _logits_body,
        grid=(chunk // bt,),
        in_specs=[
            pl.BlockSpec((bt, D), lambda i: (i, 0)),
            pl.BlockSpec((D, E), lambda i: (0, 0)),
        ],
        out_specs=pl.BlockSpec((E, bt), lambda i: (0, i)),
        out_shape=jax.ShapeDtypeStruct((E, chunk), jnp.float32),
    )(x2, kernel_DE)
    w_sc, i_sc = sc_router(lt)

    tb = chunk // bt
    w_f, i_f = pl.pallas_call(
        _fused_body,
        grid=(tail // bt,),
        in_specs=[
            pl.BlockSpec((bt, D), lambda i: (tb + i, 0)),
            pl.BlockSpec((D, E), lambda i: (0, 0)),
        ],
        out_specs=[
            pl.BlockSpec((_K, bt), lambda i: (0, i)),
            pl.BlockSpec((_K, bt), lambda i: (0, i)),
        ],
        out_shape=[
            jax.ShapeDtypeStruct((_K, tail), jnp.float32),
            jax.ShapeDtypeStruct((_K, tail), jnp.int32),
        ],
    )(x2, kernel_DE)

    w_t = jnp.concatenate([w_sc, w_f], axis=1)
    i_t = jnp.concatenate([i_sc, i_f], axis=1)
    return w_t.T.reshape(B, T, _K), i_t.T.reshape(B, T, _K)


# speedup vs baseline: 1.1695x; 1.0001x over previous
import functools
import jax
import jax.numpy as jnp
from jax import lax
from jax.experimental import pallas as pl
from jax.experimental.pallas import tpu as pltpu
from jax.experimental.pallas import tpu_sc as plsc

_K = 8
_L = 16


def _logits_body(x_ref, w_ref, out_ref):
    logits = jnp.dot(x_ref[0], w_ref[...], preferred_element_type=jnp.float32)
    out_ref[...] = logits.T


def _fused_body(x_ref, w_ref, out_w_ref, out_i_ref):
    logits = jnp.dot(x_ref[0], w_ref[...], preferred_element_type=jnp.float32)
    bt, E = logits.shape
    lt = logits.T
    m = jnp.max(lt, axis=0, keepdims=True)
    e = jnp.exp(lt - m)
    z = jnp.sum(e, axis=0, keepdims=True)
    p = e / z

    iota_e = lax.broadcasted_iota(jnp.int32, (E, bt), 0)
    vals = []
    idxs = []
    cur = p
    for _ in range(_K):
        mk = jnp.max(cur, axis=0, keepdims=True)
        hit = cur == mk
        ik = jnp.min(jnp.where(hit, iota_e, E), axis=0, keepdims=True)
        vals.append(mk)
        idxs.append(ik)
        cur = jnp.where(iota_e == ik, -jnp.inf, cur)

    v = jnp.concatenate(vals, axis=0)
    i = jnp.concatenate(idxs, axis=0)
    e2 = jnp.exp(v - v[:1])
    w = e2 / jnp.sum(e2, axis=0, keepdims=True)
    out_w_ref[...] = w
    out_i_ref[...] = i


def _make_sc_router(ntok_total, E, BT):
    info = plsc.get_sparse_core_info()
    NW = info.num_cores * info.num_subcores
    ntok = ntok_total // NW
    ngrp = ntok // _L
    mesh = plsc.VectorSubcoreMesh(core_axis_name="c", subcore_axis_name="s")

    @functools.partial(
        pl.kernel,
        mesh=mesh,
        out_type=[
            jax.ShapeDtypeStruct((_K, ntok_total), jnp.float32),
            jax.ShapeDtypeStruct((_K, ntok_total), jnp.int32),
        ],
        scratch_types=[
            pltpu.VMEM((E, ntok), jnp.float32),
            pltpu.VMEM((E * _L,), jnp.float32),
            pltpu.VMEM((_K, ntok), jnp.float32),
            pltpu.VMEM((_K, ntok), jnp.int32),
        ],
        compiler_params=pltpu.CompilerParams(needs_layout_passes=False),
    )
    def sc_router(lt_hbm, out_w_hbm, out_i_hbm, lbuf, pbuf, wbuf, ibuf):
        wid = lax.axis_index("s") * info.num_cores + lax.axis_index("c")
        base = wid * ntok
        pltpu.sync_copy(lt_hbm.at[:, pl.ds(base, ntok)], lbuf)
        lane = lax.iota(jnp.int32, _L)

        ngroups8 = E // 8
        cvec = [jnp.int32(j * _L) + lane for j in range(8)]

        def group_body(g, carry):
            gbase = g * _L
            l = [lbuf[e, pl.ds(gbase, _L)] for e in range(E)]
            m = l[0]
            for e in range(1, E):
                m = jnp.maximum(m, l[e])
            ev = [jnp.exp(l[e] - m) for e in range(E)]
            z = ev[0]
            for e in range(1, E):
                z = z + ev[e]
            r = 1.0 / z
            p = [ev[e] * r for e in range(E)]
            for e in range(E):
                pbuf[pl.ds(e * _L, _L)] = p[e]

            gmax = []
            for gg in range(ngroups8):
                mg = p[gg * 8]
                for j in range(1, 8):
                    mg = jnp.maximum(mg, p[gg * 8 + j])
                gmax.append(mg)

            vk = []
            ik = []
            for k in range(_K):
                gv = list(gmax)
                gi = [jnp.full((_L,), gg, jnp.int32) for gg in range(ngroups8)]
                n = ngroups8
                while n > 1:
                    nv, ni = [], []
                    for j in range(0, n, 2):
                        ge = gv[j] >= gv[j + 1]
                        nv.append(jnp.where(ge, gv[j], gv[j + 1]))
                        ni.append(jnp.where(ge, gi[j], gi[j + 1]))
                    gv, gi, n = nv, ni, n // 2
                gwin = gi[0]
                base_flat = gwin * (8 * _L)
                mem = [
                    plsc.load_gather(pbuf, [base_flat + cvec[j]])
                    for j in range(8)
                ]
                mv = list(mem)
                mi = [jnp.full((_L,), j, jnp.int32) for j in range(8)]
                n = 8
                while n > 1:
                    nv, ni = [], []
                    for j in range(0, n, 2):
                        ge = mv[j] >= mv[j + 1]
                        nv.append(jnp.where(ge, mv[j], mv[j + 1]))
                        ni.append(jnp.where(ge, mi[j], mi[j + 1]))
                    mv, mi, n = nv, ni, n // 2
                jwin = mi[0]
                vk.append(mv[0])
                ik.append(gwin * 8 + jwin)
                plsc.store_scatter(
                    pbuf,
                    [base_flat + jwin * _L + lane],
                    jnp.full((_L,), -1.0, jnp.float32),
                )
                newm = jnp.where(jwin == 0, -1.0, mem[0])
                for j in range(1, 8):
                    newm = jnp.maximum(
                        newm, jnp.where(jwin == j, -1.0, mem[j])
                    )
                gmax = [
                    jnp.where(gwin == gg, newm, gmax[gg])
                    for gg in range(ngroups8)
                ]

            e2 = [jnp.exp(vk[k] - vk[0]) for k in range(_K)]
            s2 = e2[0]
            for k in range(1, _K):
                s2 = s2 + e2[k]
            r2 = 1.0 / s2
            for k in range(_K):
                wbuf[k, pl.ds(gbase, _L)] = e2[k] * r2
                ibuf[k, pl.ds(gbase, _L)] = ik[k]
            return carry

        lax.fori_loop(0, ngrp, group_body, 0)
        pltpu.sync_copy(wbuf, out_w_hbm.at[:, pl.ds(base, ntok)])
        pltpu.sync_copy(ibuf, out_i_hbm.at[:, pl.ds(base, ntok)])

    return sc_router


def kernel(x, kernel_DE):
    B, T, D = x.shape
    E = kernel_DE.shape[1]
    BT = B * T
    bt = 2048
    chunk = 12288
    nchunk = 1
    tail = BT - nchunk * chunk
    tpb = T // bt
    sc_router = _make_sc_router(chunk, E, BT)

    lt = pl.pallas_call(
        _logits_body,
        grid=(chunk // bt,),
        in_specs=[
            pl.BlockSpec((1, bt, D), lambda i: (i // tpb, i % tpb, 0)),
            pl.BlockSpec((D, E), lambda i: (0, 0)),
        ],
        out_specs=pl.BlockSpec((E, bt), lambda i: (0, i)),
        out_shape=jax.ShapeDtypeStruct((E, chunk), jnp.float32),
    )(x, kernel_DE)
    w_sc, i_sc = sc_router(lt)

    tb = chunk // bt
    w_f, i_f = pl.pallas_call(
        _fused_body,
        grid=(tail // bt,),
        in_specs=[
            pl.BlockSpec(
                (1, bt, D), lambda i: ((tb + i) // tpb, (tb + i) % tpb, 0)
            ),
            pl.BlockSpec((D, E), lambda i: (0, 0)),
        ],
        out_specs=[
            pl.BlockSpec((_K, bt), lambda i: (0, i)),
            pl.BlockSpec((_K, bt), lambda i: (0, i)),
        ],
        out_shape=[
            jax.ShapeDtypeStruct((_K, tail), jnp.float32),
            jax.ShapeDtypeStruct((_K, tail), jnp.int32),
        ],
    )(x, kernel_DE)

    w_t = jnp.concatenate([w_sc, w_f], axis=1)
    i_t = jnp.concatenate([i_sc, i_f], axis=1)
    return w_t.T.reshape(B, T, _K), i_t.T.reshape(B, T, _K)
